# baseline (speedup 1.0000x reference)
import functools

import jax
import jax.numpy as jnp
from jax import lax
from jax.experimental import pallas as pl
from jax.experimental.pallas import tpu as pltpu
from jax.experimental.pallas import tpu_sc as plsc

_NUM_OPS = 64
_LANES = 16
_NCHUNK = _NUM_OPS // _LANES

_mesh = plsc.VectorSubcoreMesh(
    core_axis_name="c", subcore_axis_name="s", num_cores=2, num_subcores=16
)


@functools.partial(
    pl.kernel,
    out_type=(
        jax.ShapeDtypeStruct((1,), jnp.float32),
        jax.ShapeDtypeStruct((1,), jnp.int32),
    ),
    mesh=_mesh,
    scratch_types=[
        pltpu.VMEM((_NUM_OPS,), jnp.float32),
        pltpu.VMEM((_LANES,), jnp.float32),
        pltpu.VMEM((_LANES,), jnp.int32),
    ],
)
def _router(op_hbm, w_hbm, idx_hbm, x_v, w_v, i_v):
    cid = lax.axis_index("c")
    sid = lax.axis_index("s")

    @pl.when(jnp.logical_and(cid == 0, sid == 0))
    def _():
        pltpu.sync_copy(op_hbm, x_v)
        lane = lax.iota(jnp.int32, _LANES)
        best_v = x_v[pl.ds(0, _LANES)]
        best_i = lane
        for c in range(1, _NCHUNK):
            v = x_v[pl.ds(c * _LANES, _LANES)]
            gt = v > best_v
            best_v = jnp.where(gt, v, best_v)
            best_i = jnp.where(gt, lane + c * _LANES, best_i)
        m = best_v[0]
        gidx = best_i[0]
        for l in range(1, _LANES):
            v_l = best_v[l]
            i_l = best_i[l]
            take = jnp.logical_or(
                v_l > m, jnp.logical_and(v_l == m, i_l < gidx)
            )
            m = jnp.where(take, v_l, m)
            gidx = jnp.where(take, i_l, gidx)
        s = jnp.full((_LANES,), m * 100.0, jnp.float32)
        w_v[...] = jnp.exp(s - s)
        i_v[...] = jnp.full((_LANES,), gidx, jnp.int32)
        pltpu.sync_copy(w_v.at[pl.ds(0, 1)], w_hbm)
        pltpu.sync_copy(i_v.at[pl.ds(0, 1)], idx_hbm)


def kernel(op_enc):
    return _router(op_enc)


# speedup vs baseline: 1.0904x; 1.0904x over previous
import functools

import jax
import jax.numpy as jnp
from jax import lax
from jax.experimental import pallas as pl
from jax.experimental.pallas import tpu as pltpu
from jax.experimental.pallas import tpu_sc as plsc

_NUM_OPS = 64
_LANES = 16
_NCHUNK = _NUM_OPS // _LANES

_mesh = plsc.VectorSubcoreMesh(
    core_axis_name="c", subcore_axis_name="s", num_cores=1, num_subcores=16
)


@functools.partial(
    pl.kernel,
    out_type=(
        jax.ShapeDtypeStruct((1,), jnp.float32),
        jax.ShapeDtypeStruct((1,), jnp.int32),
    ),
    mesh=_mesh,
    scratch_types=[
        pltpu.VMEM((_NUM_OPS,), jnp.float32),
        pltpu.VMEM((_LANES,), jnp.float32),
        pltpu.VMEM((_LANES,), jnp.int32),
    ],
)
def _router(op_hbm, w_hbm, idx_hbm, x_v, w_v, i_v):
    cid = lax.axis_index("c")
    sid = lax.axis_index("s")

    @pl.when(jnp.logical_and(cid == 0, sid == 0))
    def _():
        pltpu.sync_copy(op_hbm, x_v)
        lane = lax.iota(jnp.int32, _LANES)
        best_v = x_v[pl.ds(0, _LANES)]
        best_i = lane
        for c in range(1, _NCHUNK):
            v = x_v[pl.ds(c * _LANES, _LANES)]
            gt = v > best_v
            best_v = jnp.where(gt, v, best_v)
            best_i = jnp.where(gt, lane + c * _LANES, best_i)
        m = best_v[0]
        gidx = best_i[0]
        for l in range(1, _LANES):
            v_l = best_v[l]
            i_l = best_i[l]
            take = jnp.logical_or(
                v_l > m, jnp.logical_and(v_l == m, i_l < gidx)
            )
            m = jnp.where(take, v_l, m)
            gidx = jnp.where(take, i_l, gidx)
        s = jnp.full((_LANES,), m * 100.0, jnp.float32)
        w_v[...] = jnp.exp(s - s)
        i_v[...] = jnp.full((_LANES,), gidx, jnp.int32)
        pltpu.sync_copy(w_v.at[pl.ds(0, 1)], w_hbm)
        pltpu.sync_copy(i_v.at[pl.ds(0, 1)], idx_hbm)


def kernel(op_enc):
    return _router(op_enc)


# speedup vs baseline: 1.1016x; 1.0102x over previous
import functools

import jax
import jax.numpy as jnp
from jax import lax
from jax.experimental import pallas as pl
from jax.experimental.pallas import tpu as pltpu
from jax.experimental.pallas import tpu_sc as plsc

_NUM_OPS = 64
_LANES = 16
_NCHUNK = _NUM_OPS // _LANES

_mesh = plsc.VectorSubcoreMesh(
    core_axis_name="c", subcore_axis_name="s", num_cores=1, num_subcores=1
)


@functools.partial(
    pl.kernel,
    out_type=(
        jax.ShapeDtypeStruct((1,), jnp.float32),
        jax.ShapeDtypeStruct((1,), jnp.int32),
    ),
    mesh=_mesh,
    scratch_types=[
        pltpu.VMEM((_NUM_OPS,), jnp.float32),
        pltpu.VMEM((_LANES,), jnp.float32),
        pltpu.VMEM((_LANES,), jnp.int32),
        pltpu.SemaphoreType.DMA,
        pltpu.SemaphoreType.DMA,
    ],
)
def _router(op_hbm, w_hbm, idx_hbm, x_v, w_v, i_v, sem_w, sem_i):
    pltpu.sync_copy(op_hbm, x_v)
    lane = lax.iota(jnp.int32, _LANES)
    best_v = x_v[pl.ds(0, _LANES)]
    best_i = lane
    for c in range(1, _NCHUNK):
        v = x_v[pl.ds(c * _LANES, _LANES)]
        gt = v > best_v
        best_v = jnp.where(gt, v, best_v)
        best_i = jnp.where(gt, lane + c * _LANES, best_i)
    m = best_v[0]
    gidx = best_i[0]
    for l in range(1, _LANES):
        v_l = best_v[l]
        i_l = best_i[l]
        take = jnp.logical_or(v_l > m, jnp.logical_and(v_l == m, i_l < gidx))
        m = jnp.where(take, v_l, m)
        gidx = jnp.where(take, i_l, gidx)
    s = jnp.full((_LANES,), m * 100.0, jnp.float32)
    w_v[...] = jnp.exp(s - s)
    i_v[...] = jnp.full((_LANES,), gidx, jnp.int32)
    cp_w = pltpu.async_copy(w_v.at[pl.ds(0, 1)], w_hbm, sem_w)
    cp_i = pltpu.async_copy(i_v.at[pl.ds(0, 1)], idx_hbm, sem_i)
    cp_w.wait()
    cp_i.wait()


def kernel(op_enc):
    return _router(op_enc)


# speedup vs baseline: 1.1524x; 1.0461x over previous
import functools

import jax
import jax.numpy as jnp
from jax.experimental import pallas as pl
from jax.experimental.pallas import tpu as pltpu
from jax.experimental.pallas import tpu_sc as plsc

_NUM_OPS = 64

_mesh = plsc.ScalarSubcoreMesh(axis_name="c", num_cores=1)


@functools.partial(
    pl.kernel,
    out_type=(
        jax.ShapeDtypeStruct((1,), jnp.float32),
        jax.ShapeDtypeStruct((1,), jnp.int32),
    ),
    mesh=_mesh,
    scratch_types=[
        pltpu.SMEM((_NUM_OPS,), jnp.float32),
        pltpu.SMEM((1,), jnp.float32),
        pltpu.SMEM((1,), jnp.int32),
    ],
)
def _router(op_hbm, w_hbm, idx_hbm, x_s, w_s, i_s):
    pltpu.sync_copy(op_hbm, x_s)
    m = x_s[0]
    gidx = jnp.int32(0)
    for l in range(1, _NUM_OPS):
        v_l = x_s[l]
        take = v_l > m
        m = jnp.where(take, v_l, m)
        gidx = jnp.where(take, jnp.int32(l), gidx)
    w_s[0] = (m - m) + 1.0
    i_s[0] = gidx
    pltpu.sync_copy(w_s, w_hbm)
    pltpu.sync_copy(i_s, idx_hbm)


def kernel(op_enc):
    return _router(op_enc)


# speedup vs baseline: 1.1836x; 1.0271x over previous
import functools

import jax
import jax.numpy as jnp
from jax.experimental import pallas as pl
from jax.experimental.pallas import tpu as pltpu
from jax.experimental.pallas import tpu_sc as plsc

_NUM_OPS = 64

_mesh = plsc.ScalarSubcoreMesh(axis_name="c", num_cores=1)


@functools.partial(
    pl.kernel,
    out_type=(
        jax.ShapeDtypeStruct((1,), jnp.float32),
        jax.ShapeDtypeStruct((1,), jnp.int32),
    ),
    mesh=_mesh,
    scratch_types=[
        pltpu.SMEM((_NUM_OPS,), jnp.float32),
        pltpu.SMEM((1,), jnp.float32),
        pltpu.SMEM((1,), jnp.int32),
        pltpu.SemaphoreType.DMA,
        pltpu.SemaphoreType.DMA,
    ],
)
def _router(op_hbm, w_hbm, idx_hbm, x_s, w_s, i_s, sem_w, sem_i):
    w_s[0] = jnp.float32(1.0)
    cp_w = pltpu.async_copy(w_s, w_hbm, sem_w)
    pltpu.sync_copy(op_hbm, x_s)
    m = x_s[0]
    gidx = jnp.int32(0)
    for l in range(1, _NUM_OPS):
        v_l = x_s[l]
        take = v_l > m
        m = jnp.where(take, v_l, m)
        gidx = jnp.where(take, jnp.int32(l), gidx)
    i_s[0] = gidx
    cp_i = pltpu.async_copy(i_s, idx_hbm, sem_i)
    cp_w.wait()
    cp_i.wait()


def kernel(op_enc):
    return _router(op_enc)
